# baseline (speedup 1.0000x reference)
import functools

import jax
import jax.numpy as jnp
from jax import lax
from jax.experimental import pallas as pl
from jax.experimental.pallas import tpu as pltpu
from jax.experimental.pallas import tpu_sc as plsc

N = 10000
D = 128
E = 320000

NC = 2
NS = 16
NW = NC * NS
CH = 256
K = 40
KH = 24
HALVES = ((0, 24), (24, 16))
EPW = K * CH
EPAD = EPW * NW
NPAD = 10240
ROWS_PT = NPAD // NS


def _sc_agg_body(x_hbm, srcw_hbm, dstw_hbm, zrow_hbm, zdeg_hbm,
                 agg_out, deg_out,
                 idxs_v, idxd_v, rows_v, ones_v, acc_sp, deg_sp, sem):
    c = lax.axis_index("c")
    s = lax.axis_index("s")
    wid = c * NS + s
    base = wid * EPW
    pltpu.sync_copy(zrow_hbm, acc_sp.at[pl.ds(s * ROWS_PT, ROWS_PT)])
    pltpu.sync_copy(zdeg_hbm, deg_sp.at[pl.ds(s * ROWS_PT, ROWS_PT)])
    for i in range(CH // 16):
        ones_v[pl.ds(i * 16, 16)] = jnp.ones((16,), jnp.float32)
    plsc.subcore_barrier()

    def half(off, rows_n):
        pltpu.sync_copy(srcw_hbm.at[pl.ds(base + off * CH, rows_n * CH)],
                        idxs_v.at[pl.ds(0, rows_n * CH)])
        pltpu.sync_copy(dstw_hbm.at[pl.ds(base + off * CH, rows_n * CH)],
                        idxd_v.at[pl.ds(0, rows_n * CH)])

        def chunk(m, carry):
            pltpu.async_copy(x_hbm.at[idxs_v.at[pl.ds(m * CH, CH)]],
                             rows_v, sem).wait()
            pltpu.sync_copy(rows_v, acc_sp.at[idxd_v.at[pl.ds(m * CH, CH)]],
                            add=True)
            pltpu.sync_copy(ones_v, deg_sp.at[idxd_v.at[pl.ds(m * CH, CH)]],
                            add=True)
            return carry

        lax.fori_loop(0, rows_n, chunk, 0)

    for off, rows_n in HALVES:
        half(off, rows_n)
    plsc.subcore_barrier()
    pltpu.sync_copy(acc_sp.at[pl.ds(s * ROWS_PT, ROWS_PT)],
                    agg_out.at[c, pl.ds(s * ROWS_PT, ROWS_PT)])
    pltpu.sync_copy(deg_sp.at[pl.ds(s * ROWS_PT, ROWS_PT)],
                    deg_out.at[c, pl.ds(s * ROWS_PT, ROWS_PT)])


_sc_agg = pl.kernel(
    _sc_agg_body,
    mesh=plsc.VectorSubcoreMesh(core_axis_name="c", subcore_axis_name="s"),
    out_type=[
        jax.ShapeDtypeStruct((NC, NPAD, D), jnp.float32),
        jax.ShapeDtypeStruct((NC, NPAD), jnp.float32),
    ],
    scratch_types=[
        pltpu.VMEM((KH * CH,), jnp.int32),
        pltpu.VMEM((KH * CH,), jnp.int32),
        pltpu.VMEM((CH, D), jnp.float32),
        pltpu.VMEM((CH,), jnp.float32),
        pltpu.VMEM_SHARED((NPAD, D), jnp.float32),
        pltpu.VMEM_SHARED((NPAD,), jnp.float32),
        pltpu.SemaphoreType.DMA,
    ],
)


def _layer_body(relu, h_ref, p0_ref, p1_ref, d0_ref, d1_ref,
                ws_ref, wn_ref, b_ref, o_ref):
    deg = jnp.maximum(d0_ref[...] + d1_ref[...], 1.0)
    neigh = (p0_ref[0] + p1_ref[0]) / deg
    acc = jnp.dot(h_ref[...], ws_ref[...], preferred_element_type=jnp.float32)
    acc += jnp.dot(neigh, wn_ref[...], preferred_element_type=jnp.float32)
    acc += b_ref[...]
    o_ref[...] = jnp.maximum(acc, 0.0) if relu else acc


def _tc_layer(h, aggp, d0, d1, Ws, Wn, b, relu):
    R = 400
    grid = (N // R,)
    row = pl.BlockSpec((R, D), lambda i: (i, 0))
    p0 = pl.BlockSpec((1, R, D), lambda i: (0, i, 0))
    p1 = pl.BlockSpec((1, R, D), lambda i: (1, i, 0))
    col = pl.BlockSpec((R, 1), lambda i: (i, 0))
    full = pl.BlockSpec((D, D), lambda i: (0, 0))
    bspec = pl.BlockSpec((1, D), lambda i: (0, 0))
    return pl.pallas_call(
        functools.partial(_layer_body, relu),
        grid=grid,
        in_specs=[row, p0, p1, col, col, full, full, bspec],
        out_specs=row,
        out_shape=jax.ShapeDtypeStruct((N, D), jnp.float32),
    )(h, aggp, aggp, d0, d1, Ws, Wn, b.reshape(1, D))


def kernel(h, edge_index, W_self1, W_neigh1, b1, W_self2, W_neigh2, b2):
    src = edge_index[0].astype(jnp.int32)
    dst = edge_index[1].astype(jnp.int32)
    pad = EPAD - E
    trash = N + (jnp.arange(pad, dtype=jnp.int32) % (NPAD - N))
    srcw = jnp.concatenate([src, jnp.zeros((pad,), jnp.int32)])
    dstw = jnp.concatenate([dst, trash])
    zrow = jnp.zeros((ROWS_PT, D), jnp.float32)
    zdeg = jnp.zeros((ROWS_PT,), jnp.float32)

    aggp, degp = _sc_agg(h, srcw, dstw, zrow, zdeg)
    d0 = degp[0, :N, None]
    d1 = degp[1, :N, None]
    x = _tc_layer(h, aggp, d0, d1, W_self1, W_neigh1, b1, True)
    aggp2, _ = _sc_agg(x, srcw, dstw, zrow, zdeg)
    out = _tc_layer(x, aggp2, d0, d1, W_self2, W_neigh2, b2, False)
    return out
